# Initial kernel scaffold; baseline (speedup 1.0000x reference)
import jax
import jax.numpy as jnp
from jax.experimental import pallas as pl


def kernel(rnode_feats, pnode_feats, redge_feats, pedge_feats, redge_index, pedge_index, p2r, pgraph_ids, proj_W, proj_b, e1_W, e1_b, e2_W, e2_b, conv_b, gru_Wi, gru_bi, gru_Wh, gru_bh, ld1_W, ld1_b, ld2_W, ld2_b, le1_W, le1_b, le2_W, le2_b):
    raise NotImplementedError("write your pallas kernel here")


# speedup vs baseline: 1.1445x; 1.1445x over previous
import functools

import jax
import jax.numpy as jnp
from jax import lax
from jax.experimental import pallas as pl
from jax.experimental.pallas import tpu as pltpu
from jax.experimental.pallas import tpu_sc as plsc

N = 10000
E = 20000
DIN = 128
DE = 16
D = 64
DH = 32
B = 64
STEPS = 3

NP = 10240
EP = 20480
ES = 2 * EP
NS2 = 2 * NP
KDIM = (DH + 1) * D
AGG_R = NP + 128

_HI = jax.lax.Precision.HIGHEST


def _mmb(a, b):
    return lax.dot_general(a.astype(jnp.bfloat16), b.astype(jnp.bfloat16),
                           (((1,), (0,)), ((), ())),
                           preferred_element_type=jnp.float32)


def _pad_rows(x, rows):
    return jnp.pad(x, ((0, rows - x.shape[0]), (0, 0)))


def _pad_vec(x, n, val):
    return jnp.pad(x, (0, n - x.shape[0]), constant_values=val)


def _proj_body(x_ref, w_ref, b_ref, o_ref):
    o_ref[...] = jnp.maximum(_mmb(x_ref[...], w_ref[...]) + b_ref[...], 0.0)


def _proj(x, w, b):
    blk = 2048
    g = x.shape[0] // blk
    return pl.pallas_call(
        _proj_body,
        grid=(g,),
        in_specs=[
            pl.BlockSpec((blk, x.shape[1]), lambda i: (i, 0)),
            pl.BlockSpec((x.shape[1], w.shape[1]), lambda i: (0, 0)),
            pl.BlockSpec((1, w.shape[1]), lambda i: (0, 0)),
        ],
        out_specs=pl.BlockSpec((blk, w.shape[1]), lambda i: (i, 0)),
        out_shape=jax.ShapeDtypeStruct((x.shape[0], w.shape[1]), jnp.float32),
    )(x, w, b.reshape(1, -1))


def _bdot(a, b):
    return lax.dot_general(a, b, (((1,), (0,)), ((), ())),
                           preferred_element_type=jnp.float32)


def _msg_body(h_ref, z_ref, wz_ref, e2b_ref, o_ref, zhhi_ref, zhlo_ref):
    h = h_ref[...]
    zb = z_ref[...].astype(jnp.bfloat16).astype(jnp.float32)
    f32 = jnp.float32
    for j in range(DH // 2):
        piece = jnp.concatenate(
            [zb[:, 2 * j:2 * j + 1] * h, zb[:, 2 * j + 1:2 * j + 2] * h], axis=1)
        hi = piece.astype(jnp.bfloat16)
        zhhi_ref[:, j * 2 * D:(j + 1) * 2 * D] = hi
        zhlo_ref[:, j * 2 * D:(j + 1) * 2 * D] = (piece - hi.astype(f32)).astype(jnp.bfloat16)
    wz = wz_ref[...]
    m = _bdot(zhhi_ref[...], wz) + _bdot(zhlo_ref[...], wz)
    hhi = h.astype(jnp.bfloat16)
    hlo = (h - hhi.astype(f32)).astype(jnp.bfloat16)
    e2b = e2b_ref[...]
    bhi = e2b.astype(jnp.bfloat16)
    blo = (e2b - bhi.astype(f32)).astype(jnp.bfloat16)
    m = m + _bdot(hhi, bhi) + _bdot(hlo, bhi) + _bdot(hhi, blo)
    o_ref[...] = m


def _msg(hsrc, z, w2z, e2b_mat):
    blk = 512
    g = ES // blk
    return pl.pallas_call(
        _msg_body,
        grid=(g,),
        in_specs=[
            pl.BlockSpec((blk, D), lambda i: (i, 0)),
            pl.BlockSpec((blk, DH), lambda i: (i, 0)),
            pl.BlockSpec((DH * D, D), lambda i: (0, 0)),
            pl.BlockSpec((D, D), lambda i: (0, 0)),
        ],
        out_specs=pl.BlockSpec((blk, D), lambda i: (i, 0)),
        out_shape=jax.ShapeDtypeStruct((ES, D), jnp.float32),
        scratch_shapes=[pltpu.VMEM((blk, DH * D), jnp.bfloat16),
                        pltpu.VMEM((blk, DH * D), jnp.bfloat16)],
    )(hsrc, z, w2z, e2b_mat)


def _gru_body(agg_ref, h_ref, cb_ref, wi_ref, bi_ref, wh_ref, bh_ref, o_ref):
    a = jnp.maximum(agg_ref[...] + cb_ref[...], 0.0)
    h = h_ref[...]
    gi = _mmb(a, wi_ref[...]) + bi_ref[...]
    gh = _mmb(h, wh_ref[...]) + bh_ref[...]
    r = jax.nn.sigmoid(gi[:, :D] + gh[:, :D])
    zg = jax.nn.sigmoid(gi[:, D:2 * D] + gh[:, D:2 * D])
    ng = jnp.tanh(gi[:, 2 * D:] + r * gh[:, 2 * D:])
    o_ref[...] = (1.0 - zg) * ng + zg * h


def _gru(agg, h, conv_b, wi, bi, wh, bh):
    blk = 2048
    g = NS2 // blk
    return pl.pallas_call(
        _gru_body,
        grid=(g,),
        in_specs=[
            pl.BlockSpec((blk, D), lambda i: (i, 0)),
            pl.BlockSpec((blk, D), lambda i: (i, 0)),
            pl.BlockSpec((1, D), lambda i: (0, 0)),
            pl.BlockSpec((D, 3 * D), lambda i: (0, 0)),
            pl.BlockSpec((1, 3 * D), lambda i: (0, 0)),
            pl.BlockSpec((D, 3 * D), lambda i: (0, 0)),
            pl.BlockSpec((1, 3 * D), lambda i: (0, 0)),
        ],
        out_specs=pl.BlockSpec((blk, D), lambda i: (i, 0)),
        out_shape=jax.ShapeDtypeStruct((NS2, D), jnp.float32),
    )(agg, h, conv_b.reshape(1, -1), wi, bi.reshape(1, -1), wh, bh.reshape(1, -1))


def _fin_body(hp_ref, rg_ref, gid_ref, ld1w_ref, ld1b_ref, ld2w_ref, ld2b_ref,
              le1w_ref, le1b_ref, le2w_ref, le2b_ref, o_ref, acc_ref):
    i = pl.program_id(0)

    @pl.when(i == 0)
    def _():
        acc_ref[...] = jnp.zeros((B, D), jnp.float32)

    diff = hp_ref[...] - rg_ref[...]
    t = jnp.maximum(_mmb(diff, ld1w_ref[...]) + ld1b_ref[...], 0.0)
    dh = _mmb(t, ld2w_ref[...]) + ld2b_ref[...]
    gid = gid_ref[...]
    lanes = lax.broadcasted_iota(jnp.int32, (gid.shape[0], B), 1).astype(jnp.float32)
    onehot = jnp.where(gid == lanes, 1.0, 0.0)
    acc_ref[...] += lax.dot_general(onehot, dh, (((0,), (0,)), ((), ())),
                                    precision=_HI, preferred_element_type=jnp.float32)

    @pl.when(i == pl.num_programs(0) - 1)
    def _():
        eaf = acc_ref[...]
        t2 = jnp.maximum(_mmb(eaf, le1w_ref[...]) + le1b_ref[...], 0.0)
        t2b = t2.astype(jnp.bfloat16).astype(jnp.float32)
        l2b = le2w_ref[...].astype(jnp.bfloat16).astype(jnp.float32)
        ea = jnp.sum(t2b * l2b, axis=1, keepdims=True) + le2b_ref[...]
        o_ref[...] = ea


def _final(hp, rg, gidf, ld1w, ld1b, ld2w, ld2b, le1w, le1b, le2w, le2b):
    blk = 1024
    g = NP // blk
    return pl.pallas_call(
        _fin_body,
        grid=(g,),
        in_specs=[
            pl.BlockSpec((blk, D), lambda i: (i, 0)),
            pl.BlockSpec((blk, D), lambda i: (i, 0)),
            pl.BlockSpec((blk, 1), lambda i: (i, 0)),
            pl.BlockSpec((D, 2 * D), lambda i: (0, 0)),
            pl.BlockSpec((1, 2 * D), lambda i: (0, 0)),
            pl.BlockSpec((2 * D, D), lambda i: (0, 0)),
            pl.BlockSpec((1, D), lambda i: (0, 0)),
            pl.BlockSpec((D, D), lambda i: (0, 0)),
            pl.BlockSpec((1, D), lambda i: (0, 0)),
            pl.BlockSpec((1, D), lambda i: (0, 0)),
            pl.BlockSpec((1, 1), lambda i: (0, 0)),
        ],
        out_specs=pl.BlockSpec((B, 1), lambda i: (0, 0)),
        out_shape=jax.ShapeDtypeStruct((B, 1), jnp.float32),
        scratch_shapes=[pltpu.VMEM((B, D), jnp.float32)],
    )(hp, rg, gidf, ld1w, ld1b.reshape(1, -1), ld2w, ld2b.reshape(1, -1),
      le1w, le1b.reshape(1, -1), le2w.reshape(1, -1), le2b.reshape(1, 1))


def _sc_mesh():
    return plsc.VectorSubcoreMesh(core_axis_name="c", subcore_axis_name="s",
                                  num_cores=2, num_subcores=16)


def _gather_rows(table, idx3):
    c_chunks = idx3.shape[1]

    @functools.partial(
        pl.kernel,
        mesh=_sc_mesh(),
        compiler_params=pltpu.CompilerParams(use_tc_tiling_on_sc=False),
        out_type=jax.ShapeDtypeStruct((32 * c_chunks * 128, D), jnp.float32),
        scratch_types=[
            pltpu.VMEM((c_chunks, 128), jnp.int32),
            pltpu.VMEM((128, D), jnp.float32),
            pltpu.SemaphoreType.DMA,
        ],
    )
    def k(table_hbm, idx_hbm, out_hbm, idxv, rows, sem):
        cid = lax.axis_index("c")
        sid = lax.axis_index("s")
        w = sid * 2 + cid
        pltpu.sync_copy(idx_hbm.at[w], idxv)
        base = w * (c_chunks * 128)
        for j in range(c_chunks):
            pltpu.async_copy(table_hbm.at[idxv.at[j]], rows, sem).wait()
            pltpu.sync_copy(rows, out_hbm.at[pl.ds(base + j * 128, 128)])

    return k(table, idx3)


def _scatter_graphs(m, dst3, zrows):
    rows_per_tile = AGG_R // 16

    @functools.partial(
        pl.kernel,
        mesh=_sc_mesh(),
        compiler_params=pltpu.CompilerParams(use_tc_tiling_on_sc=False),
        out_type=jax.ShapeDtypeStruct((2, NP, D), jnp.float32),
        scratch_types=[
            pltpu.VMEM((10, 128), jnp.int32),
            pltpu.VMEM((128, D), jnp.float32),
            pltpu.VMEM_SHARED((AGG_R, D), jnp.float32),
        ],
    )
    def k(m_hbm, dst_hbm, z_hbm, out_hbm, idxv, mbuf, agg_sh):
        cid = lax.axis_index("c")
        sid = lax.axis_index("s")
        pltpu.sync_copy(z_hbm.at[pl.ds(sid * rows_per_tile, rows_per_tile)],
                        agg_sh.at[pl.ds(sid * rows_per_tile, rows_per_tile)])
        pltpu.sync_copy(dst_hbm.at[cid, sid], idxv)
        plsc.subcore_barrier()
        for j in range(10):
            row0 = cid * EP + sid * 1280 + j * 128
            pltpu.sync_copy(m_hbm.at[pl.ds(row0, 128)], mbuf)
            pltpu.sync_copy(mbuf, agg_sh.at[idxv.at[j]], add=True)
        plsc.subcore_barrier()
        pltpu.sync_copy(agg_sh.at[pl.ds(sid * 640, 640)],
                        out_hbm.at[cid, pl.ds(sid * 640, 640)])

    return k(m, dst3, zrows)


def kernel(rnode_feats, pnode_feats, redge_feats, pedge_feats, redge_index,
           pedge_index, p2r, pgraph_ids, proj_W, proj_b, e1_W, e1_b, e2_W,
           e2_b, conv_b, gru_Wi, gru_bi, gru_Wh, gru_bh, ld1_W, ld1_b, ld2_W,
           ld2_b, le1_W, le1_b, le2_W, le2_b):
    x = jnp.concatenate([_pad_rows(rnode_feats, NP), _pad_rows(pnode_feats, NP)], 0)
    ef = jnp.concatenate([_pad_rows(redge_feats, EP), _pad_rows(pedge_feats, EP)], 0)
    src = jnp.concatenate([
        _pad_vec(redge_index[0], EP, 0),
        _pad_vec(pedge_index[0], EP, 0) + NP,
    ]).reshape(32, 10, 128)
    dst3 = jnp.stack([
        _pad_vec(redge_index[1], EP, NP).reshape(16, 10, 128),
        _pad_vec(pedge_index[1], EP, NP).reshape(16, 10, 128),
    ])
    p2r3 = _pad_vec(p2r, 12288, 0).reshape(32, 3, 128)
    gidf = _pad_vec(pgraph_ids, NP, B).astype(jnp.float32).reshape(NP, 1)
    w2z = e2_W.astype(jnp.bfloat16).reshape(DH * D, D)
    e2b_mat = e2_b.reshape(D, D)
    zrows = jnp.zeros((AGG_R, D), jnp.float32)

    h = _proj(x, proj_W, proj_b)
    z = _proj(ef, e1_W, e1_b)
    for _ in range(STEPS):
        hsrc = _gather_rows(h, src)
        m = _msg(hsrc, z, w2z, e2b_mat)
        agg = _scatter_graphs(m, dst3, zrows)
        h = _gru(agg.reshape(NS2, D), h, conv_b, gru_Wi, gru_bi, gru_Wh, gru_bh)
    rg = _gather_rows(h, p2r3)[:NP]
    ea = _final(h[NP:], rg, gidf, ld1_W, ld1_b, ld2_W, ld2_b,
                le1_W, le1_b, le2_W, le2_b)
    return ea.reshape(-1)


# speedup vs baseline: 1.1665x; 1.0192x over previous
import functools

import jax
import jax.numpy as jnp
from jax import lax
from jax.experimental import pallas as pl
from jax.experimental.pallas import tpu as pltpu
from jax.experimental.pallas import tpu_sc as plsc

N = 10000
E = 20000
DIN = 128
DE = 16
D = 64
DH = 32
B = 64
STEPS = 3

NP = 10240
EP = 20480
ES = 2 * EP
NS2 = 2 * NP
KDIM = (DH + 1) * D
AGG_R = NP + 128

_HI = jax.lax.Precision.HIGHEST


def _mmb(a, b):
    return lax.dot_general(a.astype(jnp.bfloat16), b.astype(jnp.bfloat16),
                           (((1,), (0,)), ((), ())),
                           preferred_element_type=jnp.float32)


def _pad_rows(x, rows):
    return jnp.pad(x, ((0, rows - x.shape[0]), (0, 0)))


def _pad_vec(x, n, val):
    return jnp.pad(x, (0, n - x.shape[0]), constant_values=val)


def _proj_body(x_ref, w_ref, b_ref, o_ref):
    o_ref[...] = jnp.maximum(_mmb(x_ref[...], w_ref[...]) + b_ref[...], 0.0)


def _proj(x, w, b):
    blk = 2048
    g = x.shape[0] // blk
    return pl.pallas_call(
        _proj_body,
        grid=(g,),
        in_specs=[
            pl.BlockSpec((blk, x.shape[1]), lambda i: (i, 0)),
            pl.BlockSpec((x.shape[1], w.shape[1]), lambda i: (0, 0)),
            pl.BlockSpec((1, w.shape[1]), lambda i: (0, 0)),
        ],
        out_specs=pl.BlockSpec((blk, w.shape[1]), lambda i: (i, 0)),
        out_shape=jax.ShapeDtypeStruct((x.shape[0], w.shape[1]), jnp.float32),
    )(x, w, b.reshape(1, -1))


def _bdot(a, b):
    return lax.dot_general(a, b, (((1,), (0,)), ((), ())),
                           preferred_element_type=jnp.float32)


def _msg_body(h_ref, z_ref, wz_ref, e2b_ref, o_ref, zhhi_ref, zhlo_ref):
    h = h_ref[...]
    zb = z_ref[...].astype(jnp.bfloat16).astype(jnp.float32)
    f32 = jnp.float32
    for j in range(DH // 2):
        piece = jnp.concatenate(
            [zb[:, 2 * j:2 * j + 1] * h, zb[:, 2 * j + 1:2 * j + 2] * h], axis=1)
        hi = piece.astype(jnp.bfloat16)
        zhhi_ref[:, j * 2 * D:(j + 1) * 2 * D] = hi
        zhlo_ref[:, j * 2 * D:(j + 1) * 2 * D] = (piece - hi.astype(f32)).astype(jnp.bfloat16)
    wz = wz_ref[...]
    m = _bdot(zhhi_ref[...], wz) + _bdot(zhlo_ref[...], wz)
    hhi = h.astype(jnp.bfloat16)
    hlo = (h - hhi.astype(f32)).astype(jnp.bfloat16)
    e2b = e2b_ref[...]
    bhi = e2b.astype(jnp.bfloat16)
    blo = (e2b - bhi.astype(f32)).astype(jnp.bfloat16)
    m = m + _bdot(hhi, bhi) + _bdot(hlo, bhi) + _bdot(hhi, blo)
    o_ref[...] = m


def _msg(hsrc, z, w2z, e2b_mat):
    blk = 512
    g = ES // blk
    return pl.pallas_call(
        _msg_body,
        grid=(g,),
        in_specs=[
            pl.BlockSpec((blk, D), lambda i: (i, 0)),
            pl.BlockSpec((blk, DH), lambda i: (i, 0)),
            pl.BlockSpec((DH * D, D), lambda i: (0, 0)),
            pl.BlockSpec((D, D), lambda i: (0, 0)),
        ],
        out_specs=pl.BlockSpec((blk, D), lambda i: (i, 0)),
        out_shape=jax.ShapeDtypeStruct((ES, D), jnp.float32),
        scratch_shapes=[pltpu.VMEM((blk, DH * D), jnp.bfloat16),
                        pltpu.VMEM((blk, DH * D), jnp.bfloat16)],
    )(hsrc, z, w2z, e2b_mat)


def _gru_body(agg_ref, h_ref, cb_ref, wi_ref, bi_ref, wh_ref, bh_ref, o_ref):
    a = jnp.maximum(agg_ref[...] + cb_ref[...], 0.0)
    h = h_ref[...]
    gi = _mmb(a, wi_ref[...]) + bi_ref[...]
    gh = _mmb(h, wh_ref[...]) + bh_ref[...]
    r = jax.nn.sigmoid(gi[:, :D] + gh[:, :D])
    zg = jax.nn.sigmoid(gi[:, D:2 * D] + gh[:, D:2 * D])
    ng = jnp.tanh(gi[:, 2 * D:] + r * gh[:, 2 * D:])
    o_ref[...] = (1.0 - zg) * ng + zg * h


def _gru(agg, h, conv_b, wi, bi, wh, bh):
    blk = 2048
    g = NS2 // blk
    return pl.pallas_call(
        _gru_body,
        grid=(g,),
        in_specs=[
            pl.BlockSpec((blk, D), lambda i: (i, 0)),
            pl.BlockSpec((blk, D), lambda i: (i, 0)),
            pl.BlockSpec((1, D), lambda i: (0, 0)),
            pl.BlockSpec((D, 3 * D), lambda i: (0, 0)),
            pl.BlockSpec((1, 3 * D), lambda i: (0, 0)),
            pl.BlockSpec((D, 3 * D), lambda i: (0, 0)),
            pl.BlockSpec((1, 3 * D), lambda i: (0, 0)),
        ],
        out_specs=pl.BlockSpec((blk, D), lambda i: (i, 0)),
        out_shape=jax.ShapeDtypeStruct((NS2, D), jnp.float32),
    )(agg, h, conv_b.reshape(1, -1), wi, bi.reshape(1, -1), wh, bh.reshape(1, -1))


def _fin_body(hp_ref, rg_ref, gid_ref, ld1w_ref, ld1b_ref, ld2w_ref, ld2b_ref,
              le1w_ref, le1b_ref, le2w_ref, le2b_ref, o_ref, acc_ref):
    i = pl.program_id(0)

    @pl.when(i == 0)
    def _():
        acc_ref[...] = jnp.zeros((B, D), jnp.float32)

    diff = hp_ref[...] - rg_ref[...]
    t = jnp.maximum(_mmb(diff, ld1w_ref[...]) + ld1b_ref[...], 0.0)
    dh = _mmb(t, ld2w_ref[...]) + ld2b_ref[...]
    gid = gid_ref[...]
    lanes = lax.broadcasted_iota(jnp.int32, (gid.shape[0], B), 1).astype(jnp.float32)
    onehot = jnp.where(gid == lanes, 1.0, 0.0)
    acc_ref[...] += lax.dot_general(onehot, dh, (((0,), (0,)), ((), ())),
                                    precision=_HI, preferred_element_type=jnp.float32)

    @pl.when(i == pl.num_programs(0) - 1)
    def _():
        eaf = acc_ref[...]
        t2 = jnp.maximum(_mmb(eaf, le1w_ref[...]) + le1b_ref[...], 0.0)
        t2b = t2.astype(jnp.bfloat16).astype(jnp.float32)
        l2b = le2w_ref[...].astype(jnp.bfloat16).astype(jnp.float32)
        ea = jnp.sum(t2b * l2b, axis=1, keepdims=True) + le2b_ref[...]
        o_ref[...] = ea


def _final(hp, rg, gidf, ld1w, ld1b, ld2w, ld2b, le1w, le1b, le2w, le2b):
    blk = 1024
    g = NP // blk
    return pl.pallas_call(
        _fin_body,
        grid=(g,),
        in_specs=[
            pl.BlockSpec((blk, D), lambda i: (i, 0)),
            pl.BlockSpec((blk, D), lambda i: (i, 0)),
            pl.BlockSpec((blk, 1), lambda i: (i, 0)),
            pl.BlockSpec((D, 2 * D), lambda i: (0, 0)),
            pl.BlockSpec((1, 2 * D), lambda i: (0, 0)),
            pl.BlockSpec((2 * D, D), lambda i: (0, 0)),
            pl.BlockSpec((1, D), lambda i: (0, 0)),
            pl.BlockSpec((D, D), lambda i: (0, 0)),
            pl.BlockSpec((1, D), lambda i: (0, 0)),
            pl.BlockSpec((1, D), lambda i: (0, 0)),
            pl.BlockSpec((1, 1), lambda i: (0, 0)),
        ],
        out_specs=pl.BlockSpec((B, 1), lambda i: (0, 0)),
        out_shape=jax.ShapeDtypeStruct((B, 1), jnp.float32),
        scratch_shapes=[pltpu.VMEM((B, D), jnp.float32)],
    )(hp, rg, gidf, ld1w, ld1b.reshape(1, -1), ld2w, ld2b.reshape(1, -1),
      le1w, le1b.reshape(1, -1), le2w.reshape(1, -1), le2b.reshape(1, 1))


def _sc_mesh():
    return plsc.VectorSubcoreMesh(core_axis_name="c", subcore_axis_name="s",
                                  num_cores=2, num_subcores=16)


def _gather_rows(table, idx3):
    c_chunks = idx3.shape[1]

    @functools.partial(
        pl.kernel,
        mesh=_sc_mesh(),
        compiler_params=pltpu.CompilerParams(use_tc_tiling_on_sc=False),
        out_type=jax.ShapeDtypeStruct((32 * c_chunks * 128, D), jnp.float32),
        scratch_types=[
            pltpu.VMEM((c_chunks, 128), jnp.int32),
            pltpu.VMEM((c_chunks * 128, D), jnp.float32),
            pltpu.SemaphoreType.DMA,
        ],
    )
    def k(table_hbm, idx_hbm, out_hbm, idxv, rows, sem):
        cid = lax.axis_index("c")
        sid = lax.axis_index("s")
        w = sid * 2 + cid
        pltpu.sync_copy(idx_hbm.at[w], idxv)
        base = w * (c_chunks * 128)
        descs = []
        for j in range(c_chunks):
            descs.append(pltpu.async_copy(
                table_hbm.at[idxv.at[j]], rows.at[pl.ds(j * 128, 128)], sem))
        for dsc in descs:
            dsc.wait()
        pltpu.sync_copy(rows, out_hbm.at[pl.ds(base, c_chunks * 128)])

    return k(table, idx3)


def _scatter_graphs(m, dst3, zrows):
    rows_per_tile = AGG_R // 16

    @functools.partial(
        pl.kernel,
        mesh=_sc_mesh(),
        compiler_params=pltpu.CompilerParams(use_tc_tiling_on_sc=False),
        out_type=jax.ShapeDtypeStruct((2, NP, D), jnp.float32),
        scratch_types=[
            pltpu.VMEM((10, 128), jnp.int32),
            pltpu.VMEM((1280, D), jnp.float32),
            pltpu.VMEM_SHARED((AGG_R, D), jnp.float32),
        ],
    )
    def k(m_hbm, dst_hbm, z_hbm, out_hbm, idxv, mbuf, agg_sh):
        cid = lax.axis_index("c")
        sid = lax.axis_index("s")
        pltpu.sync_copy(z_hbm.at[pl.ds(sid * rows_per_tile, rows_per_tile)],
                        agg_sh.at[pl.ds(sid * rows_per_tile, rows_per_tile)])
        pltpu.sync_copy(dst_hbm.at[cid, sid], idxv)
        pltpu.sync_copy(m_hbm.at[pl.ds(cid * EP + sid * 1280, 1280)], mbuf)
        plsc.subcore_barrier()
        for j in range(10):
            pltpu.sync_copy(mbuf.at[pl.ds(j * 128, 128)],
                            agg_sh.at[idxv.at[j]], add=True)
        plsc.subcore_barrier()
        pltpu.sync_copy(agg_sh.at[pl.ds(sid * 640, 640)],
                        out_hbm.at[cid, pl.ds(sid * 640, 640)])

    return k(m, dst3, zrows)


def kernel(rnode_feats, pnode_feats, redge_feats, pedge_feats, redge_index,
           pedge_index, p2r, pgraph_ids, proj_W, proj_b, e1_W, e1_b, e2_W,
           e2_b, conv_b, gru_Wi, gru_bi, gru_Wh, gru_bh, ld1_W, ld1_b, ld2_W,
           ld2_b, le1_W, le1_b, le2_W, le2_b):
    x = jnp.concatenate([_pad_rows(rnode_feats, NP), _pad_rows(pnode_feats, NP)], 0)
    ef = jnp.concatenate([_pad_rows(redge_feats, EP), _pad_rows(pedge_feats, EP)], 0)
    src = jnp.concatenate([
        _pad_vec(redge_index[0], EP, 0),
        _pad_vec(pedge_index[0], EP, 0) + NP,
    ]).reshape(32, 10, 128)
    dst3 = jnp.stack([
        _pad_vec(redge_index[1], EP, NP).reshape(16, 10, 128),
        _pad_vec(pedge_index[1], EP, NP).reshape(16, 10, 128),
    ])
    p2r3 = _pad_vec(p2r, 12288, 0).reshape(32, 3, 128)
    gidf = _pad_vec(pgraph_ids, NP, B).astype(jnp.float32).reshape(NP, 1)
    w2z = e2_W.astype(jnp.bfloat16).reshape(DH * D, D)
    e2b_mat = e2_b.reshape(D, D)
    zrows = jnp.zeros((AGG_R, D), jnp.float32)

    h = _proj(x, proj_W, proj_b)
    z = _proj(ef, e1_W, e1_b)
    for _ in range(STEPS):
        hsrc = _gather_rows(h, src)
        m = _msg(hsrc, z, w2z, e2b_mat)
        agg = _scatter_graphs(m, dst3, zrows)
        h = _gru(agg.reshape(NS2, D), h, conv_b, gru_Wi, gru_bi, gru_Wh, gru_bh)
    rg = _gather_rows(h, p2r3)[:NP]
    ea = _final(h[NP:], rg, gidf, ld1_W, ld1_b, ld2_W, ld2_b,
                le1_W, le1_b, le2_W, le2_b)
    return ea.reshape(-1)
